# baseline (speedup 1.0000x reference)
import functools

import jax
import jax.numpy as jnp
from jax.experimental import pallas as pl
from jax.experimental.pallas import tpu as pltpu

HIDDEN = 128
NGRAPHS = 1024


def _bn(x, g, b):
    return x / jnp.sqrt(1.0 + 1e-5) * g + b


def _ln(x, g, b):
    m = jnp.mean(x, axis=-1, keepdims=True)
    v = jnp.var(x, axis=-1, keepdims=True)
    return (x - m) / jnp.sqrt(v + 1e-5) * g + b


def _encoder(p, x, edge_index, batch):
    x = x @ p['We'] + p['be']
    src = edge_index[0]
    dst = edge_index[1]
    n = x.shape[0]
    for lp in p['layers']:
        agg = jnp.zeros_like(x).at[dst].add(x[src])
        h = (1.0 + lp['eps']) * x + agg
        h = jax.nn.relu(_bn(h @ lp['W1'] + lp['b1'], lp['g1'], lp['c1']))
        h = h @ lp['W2'] + lp['b2']
        x = jax.nn.relu(_bn(h, lp['g2'], lp['c2']))
    sums = jax.ops.segment_sum(x, batch, num_segments=NGRAPHS)
    cnts = jax.ops.segment_sum(jnp.ones((n,), dtype=x.dtype), batch, num_segments=NGRAPHS)
    return sums / jnp.maximum(cnts, 1.0)[:, None]


def _head_body(aq_ref, inter_ref, t_ref, wt1_ref, ct1_ref, wt2_ref, ct2_ref,
               wi_ref, ci_ref, gi_ref, bi_ref,
               wf1_ref, cf1_ref, gf1_ref, bf1_ref,
               wf2_ref, cf2_ref, gf2_ref, bf2_ref,
               wf3_ref, cf3_ref, out_ref):
    def ln(x, g, b):
        m = jnp.mean(x, axis=-1, keepdims=True)
        v = jnp.mean((x - m) ** 2, axis=-1, keepdims=True)
        return (x - m) * jax.lax.rsqrt(v + 1e-5) * g + b

    inter = inter_ref[...]
    h = ln(inter @ wi_ref[...] + ci_ref[...], gi_ref[...], bi_ref[...])
    inter_emb = jnp.maximum(h, 0.0)
    t = jnp.maximum(t_ref[...] @ wt1_ref[...] + ct1_ref[...], 0.0)
    temp_emb = t @ wt2_ref[...] + ct2_ref[...]
    fusion = jnp.concatenate([aq_ref[...], inter_emb, temp_emb], axis=1)
    h = jnp.maximum(ln(fusion @ wf1_ref[...] + cf1_ref[...], gf1_ref[...], bf1_ref[...]), 0.0)
    h = jnp.maximum(ln(h @ wf2_ref[...] + cf2_ref[...], gf2_ref[...], bf2_ref[...]), 0.0)
    out_ref[...] = h @ wf3_ref[...] + cf3_ref[...]


@jax.jit
def _head(drug_emb_aq, drug_emb_bs, solvent_emb, temperature, params):
    inter_in = jnp.concatenate([drug_emb_bs, solvent_emb], axis=1)
    args = (drug_emb_aq, inter_in, temperature,
            params['Wt1'], params['ct1'][None, :], params['Wt2'], params['ct2'][None, :],
            params['Wi'], params['ci'][None, :], params['gi'][None, :], params['bi'][None, :],
            params['Wf1'], params['cf1'][None, :], params['gf1'][None, :], params['bf1'][None, :],
            params['Wf2'], params['cf2'][None, :], params['gf2'][None, :], params['bf2'][None, :],
            params['Wf3'], params['cf3'][None, :])
    return pl.pallas_call(
        _head_body,
        out_shape=jax.ShapeDtypeStruct((NGRAPHS, 1), jnp.float32),
    )(*args)


def kernel(drug_x, drug_edge_index, drug_batch, solvent_x, solvent_edge_index,
           solvent_batch, temperature, params):
    drug_emb_aq = _encoder(params['enc_aq'], drug_x, drug_edge_index, drug_batch)
    drug_emb_bs = _encoder(params['enc_bs'], drug_x, drug_edge_index, drug_batch)
    solvent_emb = _encoder(params['enc_sol'], solvent_x, solvent_edge_index, solvent_batch)
    return _head(drug_emb_aq, drug_emb_bs, solvent_emb, temperature, params)


# speedup vs baseline: 2.9480x; 2.9480x over previous
import functools

import jax
import jax.numpy as jnp
from jax import lax
from jax.experimental import pallas as pl
from jax.experimental.pallas import tpu as pltpu, tpu_sc as plsc

HIDDEN = 128
NGRAPHS = 1024
N_NODES = 50000
N_EDGES = 800000
NP = 50048
NT = 16
NSL = 8
SLW = 16
BM = 1088
GRID = NP // BM
EPT = N_EDGES // NT
EW = 2000
NWIN = EPT // EW
RPT = NP // NT
NGP = 1040
PW = 184


def _embed_body(xp_ref, we_ref, be_ref, x_ref):
    x_ref[...] = xp_ref[...] @ we_ref[...] + be_ref[...]


def _embed(xpad, wep, be):
    return pl.pallas_call(
        _embed_body,
        grid=(GRID,),
        in_specs=[
            pl.BlockSpec((BM, 128), lambda i: (i, 0)),
            pl.BlockSpec((128, 128), lambda i: (0, 0)),
            pl.BlockSpec((1, 128), lambda i: (0, 0)),
        ],
        out_specs=pl.BlockSpec((BM, 128), lambda i: (i, 0)),
        out_shape=jax.ShapeDtypeStruct((NP, 128), jnp.float32),
    )(xpad, wep, be)


def _mlp_body(x_ref, agg_ref, eps_ref, w1_ref, b1_ref, w2_ref, b2_ref,
              x_out_ref):
    h = x_ref[...] * eps_ref[...] + agg_ref[...]
    h = jnp.maximum(h @ w1_ref[...] + b1_ref[...], 0.0)
    x_out_ref[...] = jnp.maximum(h @ w2_ref[...] + b2_ref[...], 0.0)


def _mlp(x, agg, epsv, w1f, b1f, w2f, b2f):
    return pl.pallas_call(
        _mlp_body,
        grid=(GRID,),
        in_specs=[
            pl.BlockSpec((BM, 128), lambda i: (i, 0)),
            pl.BlockSpec((BM, 128), lambda i: (i, 0)),
            pl.BlockSpec((1, 128), lambda i: (0, 0)),
            pl.BlockSpec((128, 256), lambda i: (0, 0)),
            pl.BlockSpec((1, 256), lambda i: (0, 0)),
            pl.BlockSpec((256, 128), lambda i: (0, 0)),
            pl.BlockSpec((1, 128), lambda i: (0, 0)),
        ],
        out_specs=pl.BlockSpec((BM, 128), lambda i: (i, 0)),
        out_shape=jax.ShapeDtypeStruct((NP, 128), jnp.float32),
    )(x, agg, epsv, w1f, b1f, w2f, b2f)


def _agg_body(xlin_hbm, src_hbm, dst_hbm, out_hbm,
              idxs_v, idxd_v, idx2_v, rows_v, zero_v, acc_sh, sem):
    c = lax.axis_index("c")
    s = lax.axis_index("s")
    base = s * EPT
    zero_v[...] = jnp.zeros_like(zero_v)
    for k in range(NSL):
        @pl.when(c == k // 4)
        def _round(k=k):
            for z in range(4):
                pltpu.sync_copy(zero_v, acc_sh.at[pl.ds(s * RPT + z * (RPT // 4), RPT // 4)])
            plsc.subcore_barrier()

            def win(w, carry):
                off = base + w * EW
                pltpu.sync_copy(src_hbm.at[pl.ds(off, EW)], idxs_v)
                pltpu.sync_copy(dst_hbm.at[pl.ds(off, EW)], idxd_v)

                def xform(i, carry2):
                    idx2_v[pl.ds(i * 16, 16)] = idxs_v[pl.ds(i * 16, 16)] * 8 + k
                    return carry2

                lax.fori_loop(0, EW // 16, xform, 0)
                pltpu.async_copy(xlin_hbm.at[idx2_v], rows_v, sem).wait()
                pltpu.sync_copy(rows_v, acc_sh.at[idxd_v], add=True)
                return carry

            lax.fori_loop(0, NWIN, win, 0)
            plsc.subcore_barrier()
            pltpu.sync_copy(acc_sh.at[pl.ds(s * RPT, RPT)],
                            out_hbm.at[pl.ds(s * RPT, RPT), pl.ds(k * SLW, SLW)])


def _agg(xlin, src, dst):
    mesh = plsc.VectorSubcoreMesh(core_axis_name="c", subcore_axis_name="s")
    return pl.kernel(
        _agg_body,
        out_type=jax.ShapeDtypeStruct((NP, 128), jnp.float32),
        mesh=mesh,
        compiler_params=pltpu.CompilerParams(use_tc_tiling_on_sc=False),
        scratch_types=[
            pltpu.VMEM((EW,), jnp.int32),
            pltpu.VMEM((EW,), jnp.int32),
            pltpu.VMEM((EW,), jnp.int32),
            pltpu.VMEM((EW, SLW), jnp.float32),
            pltpu.VMEM((RPT // 4, SLW), jnp.float32),
            pltpu.VMEM_SHARED((NP, SLW), jnp.float32),
            pltpu.SemaphoreType.DMA,
        ],
    )(xlin, src, dst)


def _pool_body(x_hbm, batch_hbm, sums_hbm, cnts_hbm,
               bidx_v, rows_v, ones_v, czero_v, acc_sh, cacc_sh):
    c = lax.axis_index("c")
    s = lax.axis_index("s")
    base = s * RPT
    czero_v[...] = jnp.zeros_like(czero_v)

    @pl.when(c == 0)
    def _sums0():
        pltpu.sync_copy(czero_v, acc_sh.at[pl.ds(s * (NGP // NT), NGP // NT)])
        pltpu.sync_copy(czero_v.at[:, pl.ds(0, SLW)],
                        cacc_sh.at[pl.ds(s * (NGP // NT), NGP // NT)])
        ones_v[...] = jnp.zeros_like(ones_v) + 1.0
        plsc.subcore_barrier()

        def win(w, carry):
            off = base + w * PW
            pltpu.sync_copy(batch_hbm.at[pl.ds(off, PW)], bidx_v)
            pltpu.sync_copy(x_hbm.at[pl.ds(off, PW), pl.ds(0, 64)], rows_v)
            pltpu.sync_copy(rows_v, acc_sh.at[bidx_v], add=True)
            pltpu.sync_copy(ones_v, cacc_sh.at[bidx_v], add=True)
            return carry

        lax.fori_loop(0, 17, win, 0)
        plsc.subcore_barrier()
        pltpu.sync_copy(acc_sh.at[pl.ds(s * 64, 64)],
                        sums_hbm.at[pl.ds(s * 64, 64), pl.ds(0, 64)])
        pltpu.sync_copy(cacc_sh.at[pl.ds(s * 64, 64)], cnts_hbm.at[pl.ds(s * 64, 64)])

    @pl.when(c == 1)
    def _sums1():
        pltpu.sync_copy(czero_v, acc_sh.at[pl.ds(s * (NGP // NT), NGP // NT)])
        plsc.subcore_barrier()

        def win(w, carry):
            off = base + w * PW
            pltpu.sync_copy(batch_hbm.at[pl.ds(off, PW)], bidx_v)
            pltpu.sync_copy(x_hbm.at[pl.ds(off, PW), pl.ds(64, 64)], rows_v)
            pltpu.sync_copy(rows_v, acc_sh.at[bidx_v], add=True)
            return carry

        lax.fori_loop(0, 17, win, 0)
        plsc.subcore_barrier()
        pltpu.sync_copy(acc_sh.at[pl.ds(s * 64, 64)],
                        sums_hbm.at[pl.ds(s * 64, 64), pl.ds(64, 64)])


def _pool(x, batchp):
    mesh = plsc.VectorSubcoreMesh(core_axis_name="c", subcore_axis_name="s")
    return pl.kernel(
        _pool_body,
        out_type=[
            jax.ShapeDtypeStruct((NGRAPHS, 128), jnp.float32),
            jax.ShapeDtypeStruct((NGRAPHS, SLW), jnp.float32),
        ],
        mesh=mesh,
        compiler_params=pltpu.CompilerParams(use_tc_tiling_on_sc=False),
        scratch_types=[
            pltpu.VMEM((PW,), jnp.int32),
            pltpu.VMEM((PW, 64), jnp.float32),
            pltpu.VMEM((PW, SLW), jnp.float32),
            pltpu.VMEM((NGP // NT, 64), jnp.float32),
            pltpu.VMEM_SHARED((NGP, 64), jnp.float32),
            pltpu.VMEM_SHARED((NGP, SLW), jnp.float32),
        ],
    )(x, batchp)


def _head_body(saq_ref, sbs_ref, ssol_ref, cd_ref, cs_ref, t_ref,
               wt1_ref, ct1_ref, wt2_ref, ct2_ref,
               wi_ref, ci_ref, gi_ref, bi_ref,
               wf1_ref, cf1_ref, gf1_ref, bf1_ref,
               wf2_ref, cf2_ref, gf2_ref, bf2_ref,
               wf3_ref, cf3_ref, out_ref):
    def ln(x, g, b):
        m = jnp.mean(x, axis=-1, keepdims=True)
        v = jnp.mean((x - m) ** 2, axis=-1, keepdims=True)
        return (x - m) * jax.lax.rsqrt(v + 1e-5) * g + b

    cd = jnp.maximum(cd_ref[...][:, :1], 1.0)
    cs = jnp.maximum(cs_ref[...][:, :1], 1.0)
    aq = saq_ref[...] / cd
    bs = sbs_ref[...] / cd
    sol = ssol_ref[...] / cs
    inter = jnp.concatenate([bs, sol], axis=1)
    h = ln(inter @ wi_ref[...] + ci_ref[...], gi_ref[...], bi_ref[...])
    inter_emb = jnp.maximum(h, 0.0)
    t = jnp.maximum(t_ref[...] @ wt1_ref[...] + ct1_ref[...], 0.0)
    temp_emb = t @ wt2_ref[...] + ct2_ref[...]
    fusion = jnp.concatenate([aq, inter_emb, temp_emb], axis=1)
    h = jnp.maximum(ln(fusion @ wf1_ref[...] + cf1_ref[...], gf1_ref[...], bf1_ref[...]), 0.0)
    h = jnp.maximum(ln(h @ wf2_ref[...] + cf2_ref[...], gf2_ref[...], bf2_ref[...]), 0.0)
    out_ref[...] = h @ wf3_ref[...] + cf3_ref[...]


def _head(saq, sbs, ssol, cd, cs, temperature, p):
    args = (saq, sbs, ssol, cd, cs, temperature,
            p['Wt1'], p['ct1'][None, :], p['Wt2'], p['ct2'][None, :],
            p['Wi'], p['ci'][None, :], p['gi'][None, :], p['bi'][None, :],
            p['Wf1'], p['cf1'][None, :], p['gf1'][None, :], p['bf1'][None, :],
            p['Wf2'], p['cf2'][None, :], p['gf2'][None, :], p['bf2'][None, :],
            p['Wf3'], p['cf3'][None, :])
    return pl.pallas_call(
        _head_body,
        out_shape=jax.ShapeDtypeStruct((NGRAPHS, 1), jnp.float32),
    )(*args)


xpad_feats = 9
_BN_S = (1.0 + 1e-5) ** -0.5


def _fold(w, b, g, c):
    s = g * _BN_S
    return w * s[None, :], (b * s + c)[None, :]


def _encoder(p, xpad, src, dst, batchp):
    wep = jnp.zeros((128, 128), jnp.float32).at[:xpad_feats].set(p['We'])
    x = _embed(xpad, wep, p['be'][None, :])
    for lp in p['layers']:
        agg = _agg(x.reshape(NP * NSL, SLW), src, dst)
        w1f, b1f = _fold(lp['W1'], lp['b1'], lp['g1'], lp['c1'])
        w2f, b2f = _fold(lp['W2'], lp['b2'], lp['g2'], lp['c2'])
        epsv = jnp.broadcast_to(1.0 + lp['eps'], (1, 128)).astype(jnp.float32)
        x = _mlp(x, agg, epsv, w1f, b1f, w2f, b2f)
    return _pool(x, batchp)


def kernel(drug_x, drug_edge_index, drug_batch, solvent_x, solvent_edge_index,
           solvent_batch, temperature, params):
    dxp = jnp.zeros((NP, 128), jnp.float32).at[:N_NODES, :xpad_feats].set(drug_x)
    sxp = jnp.zeros((NP, 128), jnp.float32).at[:N_NODES, :xpad_feats].set(solvent_x)
    dsrc = drug_edge_index[0].astype(jnp.int32)
    ddst = drug_edge_index[1].astype(jnp.int32)
    ssrc = solvent_edge_index[0].astype(jnp.int32)
    sdst = solvent_edge_index[1].astype(jnp.int32)
    dbat = jnp.pad(drug_batch.astype(jnp.int32), (0, NP - N_NODES),
                   constant_values=NGRAPHS)
    sbat = jnp.pad(solvent_batch.astype(jnp.int32), (0, NP - N_NODES),
                   constant_values=NGRAPHS)
    saq, cd = _encoder(params['enc_aq'], dxp, dsrc, ddst, dbat)
    sbs, _ = _encoder(params['enc_bs'], dxp, dsrc, ddst, dbat)
    ssol, cs = _encoder(params['enc_sol'], sxp, ssrc, sdst, sbat)
    return _head(saq, sbs, ssol, cd, cs, temperature, params)
